# baseline (speedup 1.0000x reference)
import functools

import jax
import jax.numpy as jnp
from jax import lax
from jax.experimental import pallas as pl
from jax.experimental.pallas import tpu as pltpu
from jax.experimental.pallas import tpu_sc as plsc

NN = 50000
NE = 800000
HID = 64

NC = 2
NS = 16
NW = NC * NS

CHUNK = 128
NCHG = 196
EPAD = NW * NCHG * CHUNK
NCHS = (EPAD // CHUNK) // NS
HALF = NN // 2
HALF_PAD = 25088
TRASH = 25080
ROWS_PER_TILE = HALF_PAD // NS
WCHUNK = 49
SCH = 64
NCHS2 = EPAD // SCH // NS
IDXG = 16

_mesh = plsc.VectorSubcoreMesh(
    core_axis_name="c", subcore_axis_name="s", num_cores=NC, num_subcores=NS)


def _sc_gather_body(a_hbm, b_hbm, dst_hbm, src_hbm, out_hbm,
                    idxd_all, idxs_all,
                    bufa0, bufb0, obuf0, bufa1, bufb1, obuf1,
                    semg0, semo0, semg1, semo1):
    w = lax.axis_index("c") * NS + lax.axis_index("s")
    base_w = w * (NCHG * CHUNK)
    pltpu.sync_copy(dst_hbm.at[pl.ds(base_w, NCHG * CHUNK)], idxd_all)
    pltpu.sync_copy(src_hbm.at[pl.ds(base_w, NCHG * CHUNK)], idxs_all)

    banks = ((bufa0, bufb0, obuf0, semg0, semo0),
             (bufa1, bufb1, obuf1, semg1, semo1))

    for b in range(2):
        bufa, bufb, _, semg, _ = banks[b]
        pltpu.async_copy(a_hbm.at[idxd_all.at[pl.ds(b * CHUNK, CHUNK)]], bufa, semg)
        pltpu.async_copy(b_hbm.at[idxs_all.at[pl.ds(b * CHUNK, CHUNK)]], bufb, semg)

    def pair(t, carry):
        for b in range(2):
            bufa, bufb, obuf, semg, semo = banks[b]
            c = 2 * t + b

            @pl.when(t > 0)
            def _():
                pltpu.make_async_copy(
                    obuf, out_hbm.at[pl.ds(0, CHUNK)], semo).wait()

            pltpu.make_async_copy(a_hbm.at[pl.ds(0, CHUNK)], bufa, semg).wait()
            pltpu.make_async_copy(a_hbm.at[pl.ds(0, CHUNK)], bufb, semg).wait()

            def add_row(r, c2):
                for cb in range(HID // 16):
                    sl = pl.ds(cb * 16, 16)
                    obuf[r, sl] = bufa[r, sl] + bufb[r, sl]
                return c2
            lax.fori_loop(0, CHUNK, add_row, 0)
            pltpu.async_copy(
                obuf, out_hbm.at[pl.ds(base_w + c * CHUNK, CHUNK)], semo)

            @pl.when(c + 2 < NCHG)
            def _():
                i0 = (c + 2) * CHUNK
                pltpu.async_copy(
                    a_hbm.at[idxd_all.at[pl.ds(i0, CHUNK)]], bufa, semg)
                pltpu.async_copy(
                    b_hbm.at[idxs_all.at[pl.ds(i0, CHUNK)]], bufb, semg)
        return carry
    lax.fori_loop(0, NCHG // 2, pair, 0)

    for b in range(2):
        _, _, obuf, _, semo = banks[b]
        pltpu.make_async_copy(obuf, out_hbm.at[pl.ds(0, CHUNK)], semo).wait()


@functools.partial(
    pl.kernel,
    out_type=jax.ShapeDtypeStruct((EPAD, HID), jnp.float32),
    mesh=_mesh,
    compiler_params=pltpu.CompilerParams(use_tc_tiling_on_sc=False),
    scratch_types=[
        pltpu.VMEM((NCHG * CHUNK,), jnp.int32),
        pltpu.VMEM((NCHG * CHUNK,), jnp.int32),
        pltpu.VMEM((CHUNK, HID), jnp.float32),
        pltpu.VMEM((CHUNK, HID), jnp.float32),
        pltpu.VMEM((CHUNK, HID), jnp.float32),
        pltpu.VMEM((CHUNK, HID), jnp.float32),
        pltpu.VMEM((CHUNK, HID), jnp.float32),
        pltpu.VMEM((CHUNK, HID), jnp.float32),
        pltpu.SemaphoreType.DMA,
        pltpu.SemaphoreType.DMA,
        pltpu.SemaphoreType.DMA,
        pltpu.SemaphoreType.DMA,
    ],
)
def _sc_gather(a_hbm, b_hbm, dst_hbm, src_hbm, out_hbm, *scratch):
    _sc_gather_body(a_hbm, b_hbm, dst_hbm, src_hbm, out_hbm, *scratch)


def _sc_scatter_body(m2_hbm, dst_hbm, out_hbm,
                     acc, idxst, il0, il1, il2, il3,
                     m2b0, m2b1, m2b2, m2b3, tmpbuf,
                     seml0, seml1, seml2, seml3,
                     sems0, sems1, sems2, sems3):
    sc = lax.axis_index("c")
    tid = lax.axis_index("s")
    nbase = sc * HALF

    z16 = jnp.zeros((16,), jnp.float32)

    def zero_row(r, c2):
        for cb in range(HID // 16):
            tmpbuf[r, pl.ds(cb * 16, 16)] = z16
        return c2
    lax.fori_loop(0, WCHUNK, zero_row, 0)
    for k in range(ROWS_PER_TILE // WCHUNK):
        pltpu.sync_copy(tmpbuf,
                        acc.at[pl.ds(tid * ROWS_PER_TILE + k * WCHUNK, WCHUNK)])
    plsc.subcore_barrier()

    base_t = tid * (NCHS2 * SCH)
    banks = ((m2b0, il0, seml0, sems0), (m2b1, il1, seml1, sems1),
             (m2b2, il2, seml2, sems2), (m2b3, il3, seml3, sems3))
    for b in range(2):
        m2b, _, seml, _ = banks[b]
        pltpu.async_copy(m2_hbm.at[pl.ds(base_t + b * SCH, SCH)], m2b, seml)

    def group(g, carry):
        pltpu.sync_copy(dst_hbm.at[pl.ds(base_t + g * IDXG * SCH,
                                         IDXG * SCH)], idxst)
        for i in range(IDXG):
            m2b, il, seml, sems = banks[i % 4]
            m2n, _, semln, semsn = banks[(i + 2) % 4]
            c = IDXG * g + i
            pltpu.make_async_copy(m2_hbm.at[pl.ds(0, SCH)], m2b, seml).wait()
            for v in range(SCH // 16):
                d = idxst[pl.ds(i * SCH + v * 16, 16)] - nbase
                ok = (d >= 0) & (d < HALF)
                il[pl.ds(v * 16, 16)] = jnp.where(ok, d, TRASH)
            pltpu.async_copy(m2b, acc.at[il], sems, add=True)

            @pl.when(c + 2 < NCHS2)
            def _():
                @pl.when(c >= 2)
                def _():
                    pltpu.make_async_copy(m2n, acc.at[pl.ds(0, SCH)],
                                          semsn).wait()
                pltpu.async_copy(
                    m2_hbm.at[pl.ds(base_t + (c + 2) * SCH, SCH)], m2n, semln)
        return carry
    lax.fori_loop(0, NCHS2 // IDXG, group, 0)
    for b in range(4):
        m2b, _, _, sems = banks[b]
        pltpu.make_async_copy(m2b, acc.at[pl.ds(0, SCH)], sems).wait()
    plsc.subcore_barrier()

    out_base = sc * HALF

    @pl.when(tid < NS - 1)
    def _():
        for k in range(ROWS_PER_TILE // WCHUNK):
            r0 = tid * ROWS_PER_TILE + k * WCHUNK
            pltpu.sync_copy(acc.at[pl.ds(r0, WCHUNK)], tmpbuf)
            pltpu.sync_copy(tmpbuf, out_hbm.at[pl.ds(out_base + r0, WCHUNK)])

    @pl.when(tid == NS - 1)
    def _():
        base15 = (NS - 1) * ROWS_PER_TILE
        nfull = (HALF - base15) // WCHUNK
        for k in range(nfull):
            r0 = base15 + k * WCHUNK
            pltpu.sync_copy(acc.at[pl.ds(r0, WCHUNK)], tmpbuf)
            pltpu.sync_copy(tmpbuf, out_hbm.at[pl.ds(out_base + r0, WCHUNK)])
        rem = HALF - base15 - nfull * WCHUNK
        r0 = base15 + nfull * WCHUNK
        pltpu.sync_copy(acc.at[pl.ds(r0, rem)], tmpbuf.at[pl.ds(0, rem)])
        pltpu.sync_copy(tmpbuf.at[pl.ds(0, rem)],
                        out_hbm.at[pl.ds(out_base + r0, rem)])


@functools.partial(
    pl.kernel,
    out_type=jax.ShapeDtypeStruct((NN, HID), jnp.float32),
    mesh=_mesh,
    compiler_params=pltpu.CompilerParams(use_tc_tiling_on_sc=False),
    scratch_types=[
        pltpu.VMEM_SHARED((HALF_PAD, HID), jnp.float32),
        pltpu.VMEM((IDXG * SCH,), jnp.int32),
        pltpu.VMEM((SCH,), jnp.int32),
        pltpu.VMEM((SCH,), jnp.int32),
        pltpu.VMEM((SCH,), jnp.int32),
        pltpu.VMEM((SCH,), jnp.int32),
        pltpu.VMEM((SCH, HID), jnp.float32),
        pltpu.VMEM((SCH, HID), jnp.float32),
        pltpu.VMEM((SCH, HID), jnp.float32),
        pltpu.VMEM((SCH, HID), jnp.float32),
        pltpu.VMEM((WCHUNK, HID), jnp.float32),
        pltpu.SemaphoreType.DMA,
        pltpu.SemaphoreType.DMA,
        pltpu.SemaphoreType.DMA,
        pltpu.SemaphoreType.DMA,
        pltpu.SemaphoreType.DMA,
        pltpu.SemaphoreType.DMA,
        pltpu.SemaphoreType.DMA,
        pltpu.SemaphoreType.DMA,
    ],
)
def _sc_scatter(m2_hbm, dst_hbm, out_hbm, *scratch):
    _sc_scatter_body(m2_hbm, dst_hbm, out_hbm, *scratch)


NP2 = NN // 2
NBLK = 5000
EBLK = 8192


def _tc_embed(x_ref, we_ref, be_ref, h_ref):
    h_ref[...] = (jnp.dot(x_ref[...], we_ref[...],
                          preferred_element_type=jnp.float32) + be_ref[...])


def _tc_ab(h_ref, wd_ref, ws_ref, a_ref, b_ref):
    hb = h_ref[...]
    a_ref[...] = jnp.dot(hb, wd_ref[...], preferred_element_type=jnp.float32)
    b_ref[...] = jnp.dot(hb, ws_ref[...], preferred_element_type=jnp.float32)


def _tc_edge(g_ref, ea_ref, w1e_ref, b1_ref, w2_ref, b2_ref, m2_ref):
    e = (g_ref[...]
         + jnp.dot(ea_ref[...], w1e_ref[...],
                   preferred_element_type=jnp.float32) + b1_ref[...])
    e = jnp.maximum(e, 0.0)
    m2 = jnp.dot(e, w2_ref[...], preferred_element_type=jnp.float32) + b2_ref[...]
    m2_ref[...] = jnp.maximum(m2, 0.0)


def _tc_node(h_ref, agg_ref, wa_ref, wb_ref, b1_ref, w2_ref, b2_ref, o_ref):
    t = (jnp.dot(h_ref[...], wa_ref[...], preferred_element_type=jnp.float32)
         + jnp.dot(agg_ref[...], wb_ref[...], preferred_element_type=jnp.float32)
         + b1_ref[...])
    t = jnp.maximum(t, 0.0)
    t = jnp.dot(t, w2_ref[...], preferred_element_type=jnp.float32) + b2_ref[...]
    o_ref[...] = jnp.maximum(t, 0.0)


def _tc_head(h_ref, w1_ref, b1_ref, w2_ref, b2_ref, o_ref, acc_ref):
    i = pl.program_id(0)

    @pl.when(i == 0)
    def _():
        acc_ref[...] = jnp.zeros_like(acc_ref)

    acc_ref[...] += jnp.sum(h_ref[...], axis=0, keepdims=True)

    @pl.when(i == (NP2 // NBLK) - 1)
    def _():
        s = acc_ref[...]
        hg = (s[:, :HID] + s[:, HID:]) * (1.0 / NN)
        z = jnp.maximum(jnp.dot(hg, w1_ref[...],
                                preferred_element_type=jnp.float32) + b1_ref[...], 0.0)
        o_ref[...] = jnp.dot(z, w2_ref[...],
                             preferred_element_type=jnp.float32) + b2_ref[...]


def _full(shape):
    return pl.BlockSpec(shape, lambda i: tuple(0 for _ in shape))


def _bd(w):
    z = jnp.zeros_like(w)
    return jnp.concatenate([jnp.concatenate([w, z], axis=1),
                            jnp.concatenate([z, w], axis=1)], axis=0)


def _b2(b):
    return jnp.concatenate([b, b]).reshape(1, 2 * HID)


def kernel(x, edge_index, edge_attr, We, be, Wem1, bem1, Wem2, bem2,
           Wnu1, bnu1, Wnu2, bnu2, Wh1, bh1, Wh2, bh2):
    src = edge_index[0]
    dst = edge_index[1]
    pad = EPAD - NE
    zpad = jnp.zeros((pad,), jnp.int32)
    dstg = jnp.concatenate([dst, zpad])
    srcg = jnp.concatenate([src, zpad])
    dsts = jnp.concatenate([dst, jnp.full((pad,), NN, jnp.int32)])
    ea2 = jnp.concatenate([edge_attr, jnp.zeros((pad, 4), jnp.float32)]
                          ).reshape(EPAD // 2, 8)

    W1d, W1s, W1e = Wem1[:HID], Wem1[HID:2 * HID], Wem1[2 * HID:]
    Wna, Wnb = Wnu1[:HID], Wnu1[HID:]
    Web = _bd(We)
    W1db, W1sb = _bd(W1d), _bd(W1s)
    W1eb = _bd(W1e)
    W2b = _bd(Wem2)
    Wnab, Wnbb = _bd(Wna), _bd(Wnb)
    Wn2b = _bd(Wnu2)
    beb, bem1b, bem2b = _b2(be), _b2(bem1), _b2(bem2)
    bnu1b, bnu2b = _b2(bnu1), _b2(bnu2)
    bh1_2 = bh1.reshape(1, HID)
    bh2_2 = bh2.reshape(1, 3)

    x2 = x.reshape(NP2, 20)
    ng = NP2 // NBLK
    nsd = pl.BlockSpec((NBLK, 2 * HID), lambda i: (i, 0))
    h = pl.pallas_call(
        _tc_embed,
        grid=(ng,),
        in_specs=[pl.BlockSpec((NBLK, 20), lambda i: (i, 0)),
                  _full((20, 2 * HID)), _full((1, 2 * HID))],
        out_specs=nsd,
        out_shape=jax.ShapeDtypeStruct((NP2, 2 * HID), jnp.float32),
    )(x2, Web, beb)

    ABLK = 4096
    nab = -(-NP2 // ABLK)
    absd = pl.BlockSpec((ABLK, 2 * HID), lambda i: (i, 0))
    abshape = jax.ShapeDtypeStruct((NP2, 2 * HID), jnp.float32)

    for rnd in range(2):
        A, B = pl.pallas_call(
            _tc_ab,
            grid=(nab,),
            in_specs=[absd, _full((2 * HID, 2 * HID)),
                      _full((2 * HID, 2 * HID))],
            out_specs=[absd, absd],
            out_shape=[abshape, abshape],
        )(h, W1db, W1sb)

        G = _sc_gather(A.reshape(NN, HID), B.reshape(NN, HID), dstg, srcg)

        M2 = pl.pallas_call(
            _tc_edge,
            grid=(EPAD // 2 // EBLK,),
            in_specs=[pl.BlockSpec((EBLK, 2 * HID), lambda i: (i, 0)),
                      pl.BlockSpec((EBLK, 8), lambda i: (i, 0)),
                      _full((8, 2 * HID)), _full((1, 2 * HID)),
                      _full((2 * HID, 2 * HID)), _full((1, 2 * HID))],
            out_specs=pl.BlockSpec((EBLK, 2 * HID), lambda i: (i, 0)),
            out_shape=jax.ShapeDtypeStruct((EPAD // 2, 2 * HID), jnp.float32),
        )(G.reshape(EPAD // 2, 2 * HID), ea2, W1eb, bem1b, W2b, bem2b)
        del A, B

        AGG = _sc_scatter(M2.reshape(EPAD, HID), dsts)
        AGG2 = AGG.reshape(NP2, 2 * HID)

        nb2 = 1000
        nsp = pl.BlockSpec((nb2, 2 * HID), lambda i: (i, 0))
        node_in = [nsp, nsp,
                   _full((2 * HID, 2 * HID)), _full((2 * HID, 2 * HID)),
                   _full((1, 2 * HID)),
                   _full((2 * HID, 2 * HID)), _full((1, 2 * HID))]
        nshape = jax.ShapeDtypeStruct((NP2, 2 * HID), jnp.float32)
        h = pl.pallas_call(
            _tc_node,
            grid=(NP2 // nb2,),
            in_specs=node_in,
            out_specs=nsp,
            out_shape=nshape,
        )(h, AGG2, Wnab, Wnbb, bnu1b, Wn2b, bnu2b)

    out = pl.pallas_call(
        _tc_head,
        grid=(ng,),
        in_specs=[pl.BlockSpec((NBLK, 2 * HID), lambda i: (i, 0)),
                  _full((HID, HID)), _full((1, HID)),
                  _full((HID, 3)), _full((1, 3))],
        out_specs=_full((1, 3)),
        out_shape=jax.ShapeDtypeStruct((1, 3), jnp.float32),
        scratch_shapes=[pltpu.VMEM((1, 2 * HID), jnp.float32)],
    )(h, Wh1, bh1_2, Wh2, bh2_2)
    return out[0]


# speedup vs baseline: 1.2479x; 1.2479x over previous
import functools

import jax
import jax.numpy as jnp
from jax import lax
from jax.experimental import pallas as pl
from jax.experimental.pallas import tpu as pltpu
from jax.experimental.pallas import tpu_sc as plsc

NN = 50000
NE = 800000
HID = 64

NC = 2
NS = 16
NW = NC * NS

CHUNK = 128
NCHG = 196
EPAD = NW * NCHG * CHUNK
NCHS = (EPAD // CHUNK) // NS
HALF = NN // 2
HALF_PAD = 25088
TRASH = 25080
ROWS_PER_TILE = HALF_PAD // NS
WCHUNK = 49
SCH = 128
NCHS2 = EPAD // SCH // NS
IDXG = 8

_mesh = plsc.VectorSubcoreMesh(
    core_axis_name="c", subcore_axis_name="s", num_cores=NC, num_subcores=NS)


def _sc_gather_body(a_hbm, b_hbm, dst_hbm, src_hbm, out_hbm,
                    idxd_all, idxs_all,
                    bufa0, bufb0, obuf0, bufa1, bufb1, obuf1,
                    semg0, semo0, semg1, semo1):
    w = lax.axis_index("c") * NS + lax.axis_index("s")
    base_w = w * (NCHG * CHUNK)
    pltpu.sync_copy(dst_hbm.at[pl.ds(base_w, NCHG * CHUNK)], idxd_all)
    pltpu.sync_copy(src_hbm.at[pl.ds(base_w, NCHG * CHUNK)], idxs_all)

    banks = ((bufa0, bufb0, obuf0, semg0, semo0),
             (bufa1, bufb1, obuf1, semg1, semo1))

    for b in range(2):
        bufa, bufb, _, semg, _ = banks[b]
        pltpu.async_copy(a_hbm.at[idxd_all.at[pl.ds(b * CHUNK, CHUNK)]], bufa, semg)
        pltpu.async_copy(b_hbm.at[idxs_all.at[pl.ds(b * CHUNK, CHUNK)]], bufb, semg)

    def pair(t, carry):
        for b in range(2):
            bufa, bufb, obuf, semg, semo = banks[b]
            c = 2 * t + b

            @pl.when(t > 0)
            def _():
                pltpu.make_async_copy(
                    obuf, out_hbm.at[pl.ds(0, CHUNK)], semo).wait()

            pltpu.make_async_copy(a_hbm.at[pl.ds(0, CHUNK)], bufa, semg).wait()
            pltpu.make_async_copy(a_hbm.at[pl.ds(0, CHUNK)], bufb, semg).wait()

            def add_row(r, c2):
                for cb in range(HID // 16):
                    sl = pl.ds(cb * 16, 16)
                    obuf[r, sl] = bufa[r, sl] + bufb[r, sl]
                return c2
            lax.fori_loop(0, CHUNK, add_row, 0)
            pltpu.async_copy(
                obuf, out_hbm.at[pl.ds(base_w + c * CHUNK, CHUNK)], semo)

            @pl.when(c + 2 < NCHG)
            def _():
                i0 = (c + 2) * CHUNK
                pltpu.async_copy(
                    a_hbm.at[idxd_all.at[pl.ds(i0, CHUNK)]], bufa, semg)
                pltpu.async_copy(
                    b_hbm.at[idxs_all.at[pl.ds(i0, CHUNK)]], bufb, semg)
        return carry
    lax.fori_loop(0, NCHG // 2, pair, 0)

    for b in range(2):
        _, _, obuf, _, semo = banks[b]
        pltpu.make_async_copy(obuf, out_hbm.at[pl.ds(0, CHUNK)], semo).wait()


@functools.partial(
    pl.kernel,
    out_type=jax.ShapeDtypeStruct((EPAD, HID), jnp.float32),
    mesh=_mesh,
    compiler_params=pltpu.CompilerParams(use_tc_tiling_on_sc=False),
    scratch_types=[
        pltpu.VMEM((NCHG * CHUNK,), jnp.int32),
        pltpu.VMEM((NCHG * CHUNK,), jnp.int32),
        pltpu.VMEM((CHUNK, HID), jnp.float32),
        pltpu.VMEM((CHUNK, HID), jnp.float32),
        pltpu.VMEM((CHUNK, HID), jnp.float32),
        pltpu.VMEM((CHUNK, HID), jnp.float32),
        pltpu.VMEM((CHUNK, HID), jnp.float32),
        pltpu.VMEM((CHUNK, HID), jnp.float32),
        pltpu.SemaphoreType.DMA,
        pltpu.SemaphoreType.DMA,
        pltpu.SemaphoreType.DMA,
        pltpu.SemaphoreType.DMA,
    ],
)
def _sc_gather(a_hbm, b_hbm, dst_hbm, src_hbm, out_hbm, *scratch):
    _sc_gather_body(a_hbm, b_hbm, dst_hbm, src_hbm, out_hbm, *scratch)


def _sc_scatter_body(m2_hbm, dst_hbm, out_hbm,
                     acc, idxst, il0, il1, m2b0, m2b1, tmpbuf,
                     seml0, seml1):
    sc = lax.axis_index("c")
    tid = lax.axis_index("s")
    nbase = sc * HALF

    z16 = jnp.zeros((16,), jnp.float32)

    def zero_row(r, c2):
        for cb in range(HID // 16):
            tmpbuf[r, pl.ds(cb * 16, 16)] = z16
        return c2
    lax.fori_loop(0, WCHUNK, zero_row, 0)
    for k in range(ROWS_PER_TILE // WCHUNK):
        pltpu.sync_copy(tmpbuf,
                        acc.at[pl.ds(tid * ROWS_PER_TILE + k * WCHUNK, WCHUNK)])
    plsc.subcore_barrier()

    base_t = tid * (NCHS2 * SCH)
    banks = ((m2b0, il0, seml0), (m2b1, il1, seml1))
    for b in range(2):
        m2b, _, seml = banks[b]
        pltpu.async_copy(m2_hbm.at[pl.ds(base_t + b * SCH, SCH)], m2b, seml)

    def group(g, carry):
        pltpu.sync_copy(dst_hbm.at[pl.ds(base_t + g * IDXG * SCH,
                                         IDXG * SCH)], idxst)
        for i in range(IDXG):
            m2b, il, seml = banks[i % 2]
            c = IDXG * g + i
            pltpu.make_async_copy(m2_hbm.at[pl.ds(0, SCH)], m2b, seml).wait()
            for v in range(SCH // 16):
                d = idxst[pl.ds(i * SCH + v * 16, 16)] - nbase
                ok = (d >= 0) & (d < HALF)
                il[pl.ds(v * 16, 16)] = jnp.where(ok, d, TRASH)
            pltpu.sync_copy(m2b, acc.at[il], add=True)

            @pl.when(c + 2 < NCHS2)
            def _():
                pltpu.async_copy(
                    m2_hbm.at[pl.ds(base_t + (c + 2) * SCH, SCH)], m2b, seml)
        return carry
    lax.fori_loop(0, NCHS2 // IDXG, group, 0)
    plsc.subcore_barrier()

    out_base = sc * HALF

    @pl.when(tid < NS - 1)
    def _():
        for k in range(ROWS_PER_TILE // WCHUNK):
            r0 = tid * ROWS_PER_TILE + k * WCHUNK
            pltpu.sync_copy(acc.at[pl.ds(r0, WCHUNK)], tmpbuf)
            pltpu.sync_copy(tmpbuf, out_hbm.at[pl.ds(out_base + r0, WCHUNK)])

    @pl.when(tid == NS - 1)
    def _():
        base15 = (NS - 1) * ROWS_PER_TILE
        nfull = (HALF - base15) // WCHUNK
        for k in range(nfull):
            r0 = base15 + k * WCHUNK
            pltpu.sync_copy(acc.at[pl.ds(r0, WCHUNK)], tmpbuf)
            pltpu.sync_copy(tmpbuf, out_hbm.at[pl.ds(out_base + r0, WCHUNK)])
        rem = HALF - base15 - nfull * WCHUNK
        r0 = base15 + nfull * WCHUNK
        pltpu.sync_copy(acc.at[pl.ds(r0, rem)], tmpbuf.at[pl.ds(0, rem)])
        pltpu.sync_copy(tmpbuf.at[pl.ds(0, rem)],
                        out_hbm.at[pl.ds(out_base + r0, rem)])


@functools.partial(
    pl.kernel,
    out_type=jax.ShapeDtypeStruct((NN, HID), jnp.float32),
    mesh=_mesh,
    compiler_params=pltpu.CompilerParams(use_tc_tiling_on_sc=False),
    scratch_types=[
        pltpu.VMEM_SHARED((HALF_PAD, HID), jnp.float32),
        pltpu.VMEM((IDXG * SCH,), jnp.int32),
        pltpu.VMEM((SCH,), jnp.int32),
        pltpu.VMEM((SCH,), jnp.int32),
        pltpu.VMEM((SCH, HID), jnp.float32),
        pltpu.VMEM((SCH, HID), jnp.float32),
        pltpu.VMEM((WCHUNK, HID), jnp.float32),
        pltpu.SemaphoreType.DMA,
        pltpu.SemaphoreType.DMA,
    ],
)
def _sc_scatter(m2_hbm, dst_hbm, out_hbm, *scratch):
    _sc_scatter_body(m2_hbm, dst_hbm, out_hbm, *scratch)


NP2 = NN // 2
NBLK = 5000
EBLK = 8192


def _tc_embed(x_ref, we_ref, be_ref, h_ref):
    h_ref[...] = (jnp.dot(x_ref[...], we_ref[...],
                          preferred_element_type=jnp.float32) + be_ref[...])


def _tc_ab(h_ref, wd_ref, ws_ref, a_ref, b_ref):
    hb = h_ref[...]
    a_ref[...] = jnp.dot(hb, wd_ref[...], preferred_element_type=jnp.float32)
    b_ref[...] = jnp.dot(hb, ws_ref[...], preferred_element_type=jnp.float32)


def _tc_edge(g_ref, ea_ref, w1e_ref, b1_ref, w2_ref, b2_ref, m2_ref):
    e = (g_ref[...]
         + jnp.dot(ea_ref[...].astype(jnp.float32), w1e_ref[...],
                   preferred_element_type=jnp.float32) + b1_ref[...])
    e = jnp.maximum(e, 0.0)
    m2 = jnp.dot(e, w2_ref[...], preferred_element_type=jnp.float32) + b2_ref[...]
    m2_ref[...] = jnp.maximum(m2, 0.0)


def _tc_node(h_ref, agg_ref, wa_ref, wb_ref, b1_ref, w2_ref, b2_ref, o_ref):
    t = (jnp.dot(h_ref[...], wa_ref[...], preferred_element_type=jnp.float32)
         + jnp.dot(agg_ref[...], wb_ref[...], preferred_element_type=jnp.float32)
         + b1_ref[...])
    t = jnp.maximum(t, 0.0)
    t = jnp.dot(t, w2_ref[...], preferred_element_type=jnp.float32) + b2_ref[...]
    o_ref[...] = jnp.maximum(t, 0.0)


def _tc_head(h_ref, w1_ref, b1_ref, w2_ref, b2_ref, o_ref, acc_ref):
    i = pl.program_id(0)

    @pl.when(i == 0)
    def _():
        acc_ref[...] = jnp.zeros_like(acc_ref)

    acc_ref[...] += jnp.sum(h_ref[...], axis=0, keepdims=True)

    @pl.when(i == (NP2 // NBLK) - 1)
    def _():
        s = acc_ref[...]
        hg = (s[:, :HID] + s[:, HID:]) * (1.0 / NN)
        z = jnp.maximum(jnp.dot(hg, w1_ref[...],
                                preferred_element_type=jnp.float32) + b1_ref[...], 0.0)
        o_ref[...] = jnp.dot(z, w2_ref[...],
                             preferred_element_type=jnp.float32) + b2_ref[...]


def _full(shape):
    return pl.BlockSpec(shape, lambda i: tuple(0 for _ in shape))


def _bd(w):
    z = jnp.zeros_like(w)
    return jnp.concatenate([jnp.concatenate([w, z], axis=1),
                            jnp.concatenate([z, w], axis=1)], axis=0)


def _b2(b):
    return jnp.concatenate([b, b]).reshape(1, 2 * HID)


def kernel(x, edge_index, edge_attr, We, be, Wem1, bem1, Wem2, bem2,
           Wnu1, bnu1, Wnu2, bnu2, Wh1, bh1, Wh2, bh2):
    src = edge_index[0]
    dst = edge_index[1]
    pad = EPAD - NE
    zpad = jnp.zeros((pad,), jnp.int32)
    dstg = jnp.concatenate([dst, zpad])
    srcg = jnp.concatenate([src, zpad])
    dsts = jnp.concatenate([dst, jnp.full((pad,), NN, jnp.int32)])
    ea2 = jnp.concatenate([edge_attr, jnp.zeros((pad, 4), jnp.float32)]
                          ).astype(jnp.bfloat16).reshape(EPAD // 2, 8)

    W1d, W1s, W1e = Wem1[:HID], Wem1[HID:2 * HID], Wem1[2 * HID:]
    Wna, Wnb = Wnu1[:HID], Wnu1[HID:]
    Web = _bd(We)
    W1db, W1sb = _bd(W1d), _bd(W1s)
    W1eb = _bd(W1e)
    W2b = _bd(Wem2)
    Wnab, Wnbb = _bd(Wna), _bd(Wnb)
    Wn2b = _bd(Wnu2)
    beb, bem1b, bem2b = _b2(be), _b2(bem1), _b2(bem2)
    bnu1b, bnu2b = _b2(bnu1), _b2(bnu2)
    bh1_2 = bh1.reshape(1, HID)
    bh2_2 = bh2.reshape(1, 3)

    x2 = x.reshape(NP2, 20)
    ng = NP2 // NBLK
    nsd = pl.BlockSpec((NBLK, 2 * HID), lambda i: (i, 0))
    h = pl.pallas_call(
        _tc_embed,
        grid=(ng,),
        in_specs=[pl.BlockSpec((NBLK, 20), lambda i: (i, 0)),
                  _full((20, 2 * HID)), _full((1, 2 * HID))],
        out_specs=nsd,
        out_shape=jax.ShapeDtypeStruct((NP2, 2 * HID), jnp.float32),
    )(x2, Web, beb)

    ABLK = 4096
    nab = -(-NP2 // ABLK)
    absd = pl.BlockSpec((ABLK, 2 * HID), lambda i: (i, 0))
    abshape = jax.ShapeDtypeStruct((NP2, 2 * HID), jnp.float32)

    for rnd in range(2):
        A, B = pl.pallas_call(
            _tc_ab,
            grid=(nab,),
            in_specs=[absd, _full((2 * HID, 2 * HID)),
                      _full((2 * HID, 2 * HID))],
            out_specs=[absd, absd],
            out_shape=[abshape, abshape],
        )(h, W1db, W1sb)

        G = _sc_gather(A.reshape(NN, HID), B.reshape(NN, HID), dstg, srcg)

        M2 = pl.pallas_call(
            _tc_edge,
            grid=(EPAD // 2 // EBLK,),
            in_specs=[pl.BlockSpec((EBLK, 2 * HID), lambda i: (i, 0)),
                      pl.BlockSpec((EBLK, 8), lambda i: (i, 0)),
                      _full((8, 2 * HID)), _full((1, 2 * HID)),
                      _full((2 * HID, 2 * HID)), _full((1, 2 * HID))],
            out_specs=pl.BlockSpec((EBLK, 2 * HID), lambda i: (i, 0)),
            out_shape=jax.ShapeDtypeStruct((EPAD // 2, 2 * HID), jnp.float32),
        )(G.reshape(EPAD // 2, 2 * HID), ea2, W1eb, bem1b, W2b, bem2b)
        del A, B

        AGG = _sc_scatter(M2.reshape(EPAD, HID), dsts)
        AGG2 = AGG.reshape(NP2, 2 * HID)

        nb2 = 1000
        nsp = pl.BlockSpec((nb2, 2 * HID), lambda i: (i, 0))
        node_in = [nsp, nsp,
                   _full((2 * HID, 2 * HID)), _full((2 * HID, 2 * HID)),
                   _full((1, 2 * HID)),
                   _full((2 * HID, 2 * HID)), _full((1, 2 * HID))]
        nshape = jax.ShapeDtypeStruct((NP2, 2 * HID), jnp.float32)
        h = pl.pallas_call(
            _tc_node,
            grid=(NP2 // nb2,),
            in_specs=node_in,
            out_specs=nsp,
            out_shape=nshape,
        )(h, AGG2, Wnab, Wnbb, bnu1b, Wn2b, bnu2b)

    out = pl.pallas_call(
        _tc_head,
        grid=(ng,),
        in_specs=[pl.BlockSpec((NBLK, 2 * HID), lambda i: (i, 0)),
                  _full((HID, HID)), _full((1, HID)),
                  _full((HID, 3)), _full((1, 3))],
        out_specs=_full((1, 3)),
        out_shape=jax.ShapeDtypeStruct((1, 3), jnp.float32),
        scratch_shapes=[pltpu.VMEM((1, 2 * HID), jnp.float32)],
    )(h, Wh1, bh1_2, Wh2, bh2_2)
    return out[0]
